# baseline (speedup 1.0000x reference)
import functools

import jax
import jax.numpy as jnp
from jax import lax
from jax.experimental import pallas as pl
from jax.experimental.pallas import tpu as pltpu
from jax.experimental.pallas import tpu_sc as plsc

_VOCAB = 101
_IDX = 128


def _bucket_of(r):
    if r == 0:
        return 0
    if r < 11:
        return 1
    if r < 37:
        return 2
    if r < 63:
        return 3
    return 4


def _make_sc_lookup(n_total, emb, chunk):
    nc, ns = 2, 16
    nw = nc * ns
    per_w = n_total // nw
    assert n_total % (nw * chunk) == 0 and chunk % _IDX == 0
    n_chunks = per_w // chunk
    g = chunk // _IDX

    mesh = plsc.VectorSubcoreMesh(
        core_axis_name="c",
        subcore_axis_name="s",
        num_cores=nc,
        num_subcores=ns,
    )

    @functools.partial(
        pl.kernel,
        mesh=mesh,
        out_type=(
            jax.ShapeDtypeStruct((n_total, emb), jnp.float32),
            jax.ShapeDtypeStruct((_VOCAB, emb), jnp.float32),
        ),
        compiler_params=pltpu.CompilerParams(
            needs_layout_passes=False, use_tc_tiling_on_sc=False
        ),
        scratch_types=[
            pltpu.VMEM((g, _IDX), jnp.int32),
            pltpu.VMEM((chunk, emb), jnp.float32),
            pltpu.VMEM((5, emb), jnp.float32),
            pltpu.VMEM((_VOCAB, emb), jnp.float32),
            pltpu.SemaphoreType.DMA,
        ],
    )
    def sc_lookup(chars_hbm, w_hbm, out_hbm, table_hbm, ids_v, rows_v, w_v,
                  tb_v, sem):
        wid = lax.axis_index("s") * nc + lax.axis_index("c")

        pltpu.sync_copy(w_hbm, w_v)
        for r in range(_VOCAB):
            tb_v[r, :] = w_v[_bucket_of(r), :]
        pltpu.sync_copy(tb_v, table_hbm)

        def chunk_body(k, _):
            row0 = (wid * n_chunks + k) * g
            pltpu.sync_copy(chars_hbm.at[pl.ds(row0, g)], ids_v)
            cps = [
                pltpu.async_copy(
                    table_hbm.at[ids_v.at[j]],
                    rows_v.at[pl.ds(j * _IDX, _IDX)],
                    sem,
                )
                for j in range(g)
            ]
            for cp in cps:
                cp.wait()
            pltpu.sync_copy(rows_v, out_hbm.at[pl.ds(row0 * _IDX, chunk)])
            return 0

        lax.fori_loop(0, n_chunks, chunk_body, 0, unroll=False)

    return sc_lookup


def kernel(chars, embedding_weight):
    b, s = chars.shape
    n_cls, emb = embedding_weight.shape
    n_total = b * s
    out, _ = _make_sc_lookup(n_total, emb, chunk=2048)(
        chars.reshape(n_total // _IDX, _IDX), embedding_weight
    )
    return out.reshape(b, s, emb)


# speedup vs baseline: 1.3518x; 1.3518x over previous
import functools

import jax
import jax.numpy as jnp
from jax import lax
from jax.experimental import pallas as pl
from jax.experimental.pallas import tpu as pltpu
from jax.experimental.pallas import tpu_sc as plsc

_L = 16
_REP = 81


def _bucket(cv):
    b = jnp.where(cv >= 1, jnp.int32(1), jnp.int32(0))
    b = jnp.where(cv >= 11, jnp.int32(2), b)
    b = jnp.where(cv >= 37, jnp.int32(3), b)
    b = jnp.where(cv >= 63, jnp.int32(4), b)
    return b


def _make_sc_lookup(n_total, emb, chunk):
    nc, ns = 2, 16
    nw = nc * ns
    per_w = n_total // nw
    assert n_total % nw == 0 and per_w % chunk == 0
    n_chunks = per_w // chunk
    groups = chunk // _L

    mesh = plsc.VectorSubcoreMesh(
        core_axis_name="c",
        subcore_axis_name="s",
        num_cores=nc,
        num_subcores=ns,
    )

    @functools.partial(
        pl.kernel,
        mesh=mesh,
        out_type=jax.ShapeDtypeStruct((n_total * emb,), jnp.float32),
        compiler_params=pltpu.CompilerParams(needs_layout_passes=False),
        scratch_types=[
            pltpu.VMEM((chunk,), jnp.int32),
            pltpu.VMEM((chunk * emb,), jnp.float32),
            pltpu.VMEM((5, emb), jnp.float32),
            pltpu.VMEM((_L * _REP,), jnp.float32),
        ],
    )
    def sc_lookup(chars_hbm, w_hbm, out_hbm, ids_v, rows_v, w_v, wt_v):
        wid = lax.axis_index("s") * nc + lax.axis_index("c")
        base = wid * per_w
        pltpu.sync_copy(w_hbm, w_v)
        for c in range(_L):
            for b in range(5):
                wt_v[pl.ds(c * _REP + b * emb, emb)] = w_v[b, :]
        lane16 = lax.iota(jnp.int32, _L) * emb
        lane81 = lax.iota(jnp.int32, _L) * _REP

        def chunk_body(k, _):
            off = base + k * chunk
            pltpu.sync_copy(chars_hbm.at[pl.ds(off, chunk)], ids_v)

            def group_body(g, _):
                cv = ids_v[pl.ds(g * _L, _L)]
                fb = _bucket(cv) * emb + lane81
                obase = g * (_L * emb) + lane16
                for e in range(emb):
                    row = plsc.load_gather(wt_v, [fb + e])
                    plsc.store_scatter(rows_v, [obase + e], row)
                return 0

            lax.fori_loop(0, groups, group_body, 0, unroll=4)
            pltpu.sync_copy(rows_v, out_hbm.at[pl.ds(off * emb, chunk * emb)])
            return 0

        lax.fori_loop(0, n_chunks, chunk_body, 0, unroll=False)

    return sc_lookup


def kernel(chars, embedding_weight):
    b, s = chars.shape
    n_cls, emb = embedding_weight.shape
    n_total = b * s
    out_flat = _make_sc_lookup(n_total, emb, chunk=2048)(
        chars.reshape(n_total), embedding_weight
    )
    return out_flat.reshape(b, s, emb)


# speedup vs baseline: 5.5035x; 4.0711x over previous
import functools

import jax
import jax.numpy as jnp
from jax import lax
from jax.experimental import pallas as pl
from jax.experimental.pallas import tpu as pltpu
from jax.experimental.pallas import tpu_sc as plsc

_L = 16
_REP = 81
_SCH = 8


def _bucket(cv):
    b = jnp.where(cv >= 1, jnp.int32(1), jnp.int32(0))
    b = jnp.where(cv >= 11, jnp.int32(2), b)
    b = jnp.where(cv >= 37, jnp.int32(3), b)
    b = jnp.where(cv >= 63, jnp.int32(4), b)
    return b


def _make_sc_lookup(bsz, seq, emb):
    nc, ns = 2, 16
    nw = nc * ns
    nbb = bsz // 128
    bb_per_w = nbb // nw
    n_sch = seq // _SCH
    eb = emb // 8

    mesh = plsc.VectorSubcoreMesh(
        core_axis_name="c",
        subcore_axis_name="s",
        num_cores=nc,
        num_subcores=ns,
    )

    @functools.partial(
        pl.kernel,
        mesh=mesh,
        out_type=jax.ShapeDtypeStruct((seq, eb, nbb, 8, 128), jnp.float32),
        compiler_params=pltpu.CompilerParams(needs_layout_passes=False),
        scratch_types=[
            pltpu.VMEM((_SCH, 128), jnp.int32),
            pltpu.VMEM((_SCH, eb, 8, 128), jnp.float32),
            pltpu.VMEM((5, emb), jnp.float32),
            pltpu.VMEM((_L * _REP,), jnp.float32),
        ],
    )
    def sc_lookup(chars_hbm, w_hbm, out_hbm, ids_v, rows_v, w_v, wt_v):
        wid = lax.axis_index("s") * nc + lax.axis_index("c")
        pltpu.sync_copy(w_hbm, w_v)
        for c in range(_L):
            for b in range(5):
                wt_v[pl.ds(c * _REP + b * emb, emb)] = w_v[b, :]
        lane81 = lax.iota(jnp.int32, _L) * _REP

        for i in range(bb_per_w):
            bb = wid * bb_per_w + i

            def chunk_body(sc, _):
                s0 = sc * _SCH
                pltpu.sync_copy(
                    chars_hbm.at[pl.ds(s0, _SCH), pl.ds(bb * 128, 128)], ids_v
                )

                def s_body(si, _):
                    for bg in range(8):
                        cv = ids_v[si, pl.ds(bg * _L, _L)]
                        fb = _bucket(cv) * emb + lane81
                        for e in range(emb):
                            row = plsc.load_gather(wt_v, [fb + e])
                            rows_v[si, e // 8, e % 8, pl.ds(bg * _L, _L)] = row
                    return 0

                lax.fori_loop(0, _SCH, s_body, 0, unroll=False)
                pltpu.sync_copy(
                    rows_v, out_hbm.at[pl.ds(s0, _SCH), :, bb]
                )
                return 0

            lax.fori_loop(0, n_sch, chunk_body, 0, unroll=False)

    return sc_lookup


def kernel(chars, embedding_weight):
    bsz, seq = chars.shape
    n_cls, emb = embedding_weight.shape
    out5 = _make_sc_lookup(bsz, seq, emb)(chars.T, embedding_weight)
    return out5.transpose(2, 4, 0, 1, 3).reshape(bsz, seq, emb)


# speedup vs baseline: 7.3199x; 1.3300x over previous
import functools

import jax
import jax.numpy as jnp
from jax import lax
from jax.experimental import pallas as pl
from jax.experimental.pallas import tpu as pltpu
from jax.experimental.pallas import tpu_sc as plsc

_L = 16
_REP = 81
_SCH = 8


def _bucket(cv):
    b = jnp.where(cv >= 1, jnp.int32(1), jnp.int32(0))
    b = jnp.where(cv >= 11, jnp.int32(2), b)
    b = jnp.where(cv >= 37, jnp.int32(3), b)
    b = jnp.where(cv >= 63, jnp.int32(4), b)
    return b


def _make_sc_lookup(bsz, seq, emb):
    nc, ns = 2, 16
    nw = nc * ns
    nbb = bsz // 128
    bb_per_w = nbb // nw
    n_sch = seq // _SCH
    n_chunks = bb_per_w * n_sch
    eb = emb // 8

    mesh = plsc.VectorSubcoreMesh(
        core_axis_name="c",
        subcore_axis_name="s",
        num_cores=nc,
        num_subcores=ns,
    )

    @functools.partial(
        pl.kernel,
        mesh=mesh,
        out_type=jax.ShapeDtypeStruct((seq, eb, nbb, 8, 128), jnp.float32),
        compiler_params=pltpu.CompilerParams(needs_layout_passes=False),
        scratch_types=[
            pltpu.VMEM((2, _SCH, 128), jnp.int32),
            pltpu.VMEM((2, _SCH, eb, 8, 128), jnp.float32),
            pltpu.VMEM((5, emb), jnp.float32),
            pltpu.VMEM((_L * _REP,), jnp.float32),
            pltpu.SemaphoreType.DMA,
            pltpu.SemaphoreType.DMA,
            pltpu.SemaphoreType.DMA,
            pltpu.SemaphoreType.DMA,
        ],
    )
    def sc_lookup(chars_hbm, w_hbm, out_hbm, ids_v, rows_v, w_v, wt_v,
                  si0, si1, so0, so1):
        wid = lax.axis_index("s") * nc + lax.axis_index("c")
        bb0 = wid * bb_per_w
        sin = (si0, si1)
        sout = (so0, so1)
        pltpu.sync_copy(w_hbm, w_v)
        for c in range(_L):
            for b in range(5):
                wt_v[pl.ds(c * _REP + b * emb, emb)] = w_v[b, :]
        lane81 = lax.iota(jnp.int32, _L) * _REP

        def in_src(k):
            s0 = (k >> 2) * _SCH
            bb = bb0 + (k & 3)
            return chars_hbm.at[pl.ds(s0, _SCH), pl.ds(bb * 128, 128)]

        def out_dst(k):
            s0 = (k >> 2) * _SCH
            bb = bb0 + (k & 3)
            return out_hbm.at[pl.ds(s0, _SCH), :, bb]

        def compute(b):
            def s_body(si, _):
                for bg in range(8):
                    cv = ids_v[b, si, pl.ds(bg * _L, _L)]
                    fb = _bucket(cv) * emb + lane81
                    for e in range(emb):
                        row = plsc.load_gather(wt_v, [fb + e])
                        rows_v[b, si, e // 8, e % 8, pl.ds(bg * _L, _L)] = row
                return 0

            lax.fori_loop(0, _SCH, s_body, 0, unroll=False)

        cp_in = [pltpu.async_copy(in_src(b), ids_v.at[b], sin[b])
                 for b in (0, 1)]

        for b in (0, 1):
            cp_in[b].wait()
            compute(b)
            pltpu.async_copy(rows_v.at[b], out_dst(b), sout[b])
            pltpu.async_copy(in_src(b + 2), ids_v.at[b], sin[b])

        def pair_body(m, _):
            for b in (0, 1):
                k = 2 * m + b
                pltpu.make_async_copy(in_src(k), ids_v.at[b], sin[b]).wait()
                pltpu.make_async_copy(rows_v.at[b], out_dst(k - 2),
                                      sout[b]).wait()
                compute(b)
                pltpu.async_copy(rows_v.at[b], out_dst(k), sout[b])

                @pl.when(k + 2 < n_chunks)
                def _():
                    pltpu.async_copy(in_src(k + 2), ids_v.at[b], sin[b])

            return 0

        lax.fori_loop(1, n_chunks // 2, pair_body, 0, unroll=False)

        for b in (0, 1):
            k = n_chunks - 2 + b
            pltpu.make_async_copy(rows_v.at[b], out_dst(k), sout[b]).wait()

    return sc_lookup


def kernel(chars, embedding_weight):
    bsz, seq = chars.shape
    n_cls, emb = embedding_weight.shape
    out5 = _make_sc_lookup(bsz, seq, emb)(chars.T, embedding_weight)
    return out5.transpose(2, 4, 0, 1, 3).reshape(bsz, seq, emb)
